# Initial kernel scaffold; baseline (speedup 1.0000x reference)
import jax
import jax.numpy as jnp
from jax import lax
from jax.experimental import pallas as pl
from jax.experimental.pallas import tpu as pltpu
from jax.experimental.pallas import tpu_sc as plsc

N = 10000
E = 320000
F = 128
R = 8

NC = 2
NS = 16
NW = NC * NS
EPW = E // NW
K = 80
NCHUNK = EPW // K
NPAD = 10240
TROWS = NPAD // NS


def _xr_kernel(x_ref, w_ref, out_ref):
    out_ref[0] = jnp.dot(x_ref[...], w_ref[0],
                         preferred_element_type=jnp.float32)


def _make_xr(x, weight):
    nb = 400
    grid = (R, N // nb)
    xr = pl.pallas_call(
        _xr_kernel,
        grid=grid,
        in_specs=[
            pl.BlockSpec((nb, F), lambda r, b: (b, 0)),
            pl.BlockSpec((1, F, F), lambda r, b: (r, 0, 0)),
        ],
        out_specs=pl.BlockSpec((1, nb, F), lambda r, b: (r, b, 0)),
        out_shape=jax.ShapeDtypeStruct((R, N, F), jnp.float32),
    )(x, weight)
    return xr.reshape(R * N, F)


def _sc_body(xr_hbm, src_hbm, dst_hbm, typ_hbm, zeros_hbm, partial_hbm,
             src_v, dst_v, typ_v, gidx_v, rows_v, acc_shared, sem):
    cid = lax.axis_index("c")
    sid = lax.axis_index("s")
    wid = sid * NC + cid

    pltpu.sync_copy(src_hbm.at[wid], src_v)
    pltpu.sync_copy(typ_hbm.at[wid], typ_v)
    pltpu.sync_copy(dst_hbm.at[wid], dst_v)

    zbase = sid * TROWS
    pltpu.sync_copy(zeros_hbm.at[pl.ds(zbase, TROWS)],
                    acc_shared.at[pl.ds(zbase, TROWS)])

    def _chunk_idx(j, _):
        def _vec(i, _):
            sl = pl.ds(i * 16, 16)
            gidx_v[j, sl] = typ_v[j, sl] * N + src_v[j, sl]
            return 0
        return lax.fori_loop(0, K // 16, _vec, 0)

    lax.fori_loop(0, NCHUNK, _chunk_idx, 0)

    plsc.subcore_barrier()

    def _chunk(j, _):
        pltpu.async_copy(xr_hbm.at[gidx_v.at[j]], rows_v, sem).wait()
        pltpu.sync_copy(rows_v, acc_shared.at[dst_v.at[j]], add=True)
        return 0

    lax.fori_loop(0, NCHUNK, _chunk, 0)

    plsc.subcore_barrier()

    pltpu.sync_copy(acc_shared.at[pl.ds(zbase, TROWS)],
                    partial_hbm.at[cid, pl.ds(zbase, TROWS)])


def _scatter_partials(xr_flat, src3, dst3, typ3, zeros):
    mesh = plsc.VectorSubcoreMesh(core_axis_name="c", subcore_axis_name="s",
                                  num_cores=NC, num_subcores=NS)
    return pl.kernel(
        _sc_body,
        out_type=jax.ShapeDtypeStruct((NC, NPAD, F), jnp.float32),
        mesh=mesh,
        scratch_types=[
            pltpu.VMEM((NCHUNK, K), jnp.int32),
            pltpu.VMEM((NCHUNK, K), jnp.int32),
            pltpu.VMEM((NCHUNK, K), jnp.int32),
            pltpu.VMEM((NCHUNK, K), jnp.int32),
            pltpu.VMEM((K, F), jnp.float32),
            pltpu.VMEM_SHARED((NPAD, F), jnp.float32),
            pltpu.SemaphoreType.DMA,
        ],
    )(xr_flat, src3, dst3, typ3, zeros)


def _combine_kernel(p_ref, b_ref, out_ref):
    out_ref[...] = p_ref[0] + p_ref[1] + b_ref[...]


def _combine(partial, h_bias):
    nb = 400
    return pl.pallas_call(
        _combine_kernel,
        grid=(N // nb,),
        in_specs=[
            pl.BlockSpec((NC, nb, F), lambda b: (0, b, 0)),
            pl.BlockSpec((1, F), lambda b: (0, 0)),
        ],
        out_specs=pl.BlockSpec((nb, F), lambda b: (b, 0)),
        out_shape=jax.ShapeDtypeStruct((N, F), jnp.float32),
    )(partial, h_bias.reshape(1, F))


@jax.jit
def kernel(x, edge_index, edge_type, weight, h_bias):
    xr_flat = _make_xr(x, weight)
    src3 = edge_index[0].reshape(NW, NCHUNK, K)
    dst3 = edge_index[1].reshape(NW, NCHUNK, K)
    typ3 = edge_type.reshape(NW, NCHUNK, K)
    zeros = jnp.zeros((NPAD, F), jnp.float32)
    partial = _scatter_partials(xr_flat, src3, dst3, typ3, zeros)
    return _combine(partial, h_bias)


# speedup vs baseline: 12.0569x; 12.0569x over previous
import jax
import jax.numpy as jnp
from jax import lax
from jax.experimental import pallas as pl
from jax.experimental.pallas import tpu as pltpu
from jax.experimental.pallas import tpu_sc as plsc

N = 10000
E = 320000
F = 128
R = 8

NC = 2
NS = 16
NW = NC * NS
EPW = E // NW
K = 128
NCHUNK = 79
EPWP = NCHUNK * K
NPAD = 10240
TRASH = N + 16
TROWS = NPAD // NS


def _xr_kernel(x_ref, w_ref, out_ref):
    out_ref[0] = jnp.dot(x_ref[...], w_ref[0],
                         preferred_element_type=jnp.float32)


def _make_xr(x, weight):
    nb = 400
    xr = pl.pallas_call(
        _xr_kernel,
        grid=(R, N // nb),
        in_specs=[
            pl.BlockSpec((nb, F), lambda r, b: (b, 0)),
            pl.BlockSpec((1, F, F), lambda r, b: (r, 0, 0)),
        ],
        out_specs=pl.BlockSpec((1, nb, F), lambda r, b: (r, b, 0)),
        out_shape=jax.ShapeDtypeStruct((R, N, F), jnp.float32),
    )(x, weight)
    return xr.reshape(R * N, F)


def _gidx_kernel(s_ref, t_ref, out_ref):
    out_ref[...] = t_ref[...] * N + s_ref[...]


def _make_gidx(src, typ):
    src2 = src.reshape(E // F, F)
    typ2 = typ.reshape(E // F, F)
    return pl.pallas_call(
        _gidx_kernel,
        out_shape=jax.ShapeDtypeStruct((E // F, F), jnp.int32),
    )(src2, typ2).reshape(E)


def _sc_body(xr_hbm, gidx_hbm, dst_hbm, zeros_hbm, partial_hbm,
             gidx_v, dst_v, rows_v, acc_shared, sem):
    cid = lax.axis_index("c")
    sid = lax.axis_index("s")
    wid = sid * NC + cid

    pltpu.sync_copy(gidx_hbm.at[wid], gidx_v)
    pltpu.sync_copy(dst_hbm.at[wid], dst_v)

    zbase = sid * TROWS
    pltpu.sync_copy(zeros_hbm.at[pl.ds(zbase, TROWS)],
                    acc_shared.at[pl.ds(zbase, TROWS)])
    plsc.subcore_barrier()

    def _chunk(j, _):
        pltpu.async_copy(xr_hbm.at[gidx_v.at[j]], rows_v, sem).wait()
        pltpu.sync_copy(rows_v, acc_shared.at[dst_v.at[j]], add=True)
        return 0

    lax.fori_loop(0, NCHUNK, _chunk, 0)
    plsc.subcore_barrier()

    pltpu.sync_copy(acc_shared.at[pl.ds(zbase, TROWS)],
                    partial_hbm.at[cid, pl.ds(zbase, TROWS)])


def _scatter_partials(xr_flat, gidx3, dst3, zeros):
    mesh = plsc.VectorSubcoreMesh(core_axis_name="c", subcore_axis_name="s",
                                  num_cores=NC, num_subcores=NS)
    return pl.kernel(
        _sc_body,
        out_type=jax.ShapeDtypeStruct((NC, NPAD, F), jnp.float32),
        mesh=mesh,
        scratch_types=[
            pltpu.VMEM((NCHUNK, K), jnp.int32),
            pltpu.VMEM((NCHUNK, K), jnp.int32),
            pltpu.VMEM((K, F), jnp.float32),
            pltpu.VMEM_SHARED((NPAD, F), jnp.float32),
            pltpu.SemaphoreType.DMA,
        ],
    )(xr_flat, gidx3, dst3, zeros)


def _combine_kernel(p_ref, b_ref, out_ref):
    out_ref[...] = p_ref[0] + p_ref[1] + b_ref[...]


def _combine(partial, h_bias):
    nb = 400
    return pl.pallas_call(
        _combine_kernel,
        grid=(N // nb,),
        in_specs=[
            pl.BlockSpec((NC, nb, F), lambda b: (0, b, 0)),
            pl.BlockSpec((1, F), lambda b: (0, 0)),
        ],
        out_specs=pl.BlockSpec((nb, F), lambda b: (b, 0)),
        out_shape=jax.ShapeDtypeStruct((N, F), jnp.float32),
    )(partial, h_bias.reshape(1, F))


def _pad_per_worker(a, fill):
    a2 = a.reshape(NW, EPW)
    a2 = jnp.pad(a2, ((0, 0), (0, EPWP - EPW)), constant_values=fill)
    return a2.reshape(NW, NCHUNK, K)


@jax.jit
def kernel(x, edge_index, edge_type, weight, h_bias):
    xr_flat = _make_xr(x, weight)
    gidx = _make_gidx(edge_index[0], edge_type)
    gidx3 = _pad_per_worker(gidx, 0)
    dst3 = _pad_per_worker(edge_index[1], TRASH)
    zeros = jnp.zeros((NPAD, F), jnp.float32)
    partial = _scatter_partials(xr_flat, gidx3, dst3, zeros)
    return _combine(partial, h_bias)
